# baseline (speedup 1.0000x reference)
import functools

import jax
import jax.numpy as jnp
from jax import lax
from jax.experimental import pallas as pl
from jax.experimental.pallas import tpu as pltpu
from jax.experimental.pallas import tpu_sc as plsc

N = 10000
E = 160000
D = 128
R = 200
TPAD = 368
NC, NS = 2, 16
K = 40
CH = 250
EPT = CH * K
EP = EPT * NS
NCHT = NS * CH
GK = 3 * K
NPAD = 10240
NROWS = NPAD // NS
TROWS = 2 * N + 2 * R + 2 * TPAD


def _mm_body(xr, wr, outr):
    outr[...] = jnp.dot(xr[...], wr[...], preferred_element_type=jnp.float32)


def _mm(x, w):
    m, k = x.shape
    n = w.shape[1]
    bm = min(m, 512)
    return pl.pallas_call(
        _mm_body,
        grid=(pl.cdiv(m, bm),),
        in_specs=[
            pl.BlockSpec((bm, k), lambda i: (i, 0)),
            pl.BlockSpec((k, n), lambda i: (0, 0)),
        ],
        out_specs=pl.BlockSpec((bm, n), lambda i: (i, 0)),
        out_shape=jax.ShapeDtypeStruct((m, n), jnp.float32),
    )(x, w)


def _blockdiag(a, b):
    z = jnp.zeros((a.shape[0], b.shape[1]), jnp.float32)
    z2 = jnp.zeros((b.shape[0], a.shape[1]), jnp.float32)
    return jnp.concatenate(
        [jnp.concatenate([a, z], 1), jnp.concatenate([z2, b], 1)], 0)


def _ileave(x):
    r, c = x.shape
    y = x.reshape(r, c // 32, 2, 16).swapaxes(2, 3).reshape(r, c // 2, 2)
    return lax.bitcast_convert_type(y.astype(jnp.bfloat16), jnp.int32)


def _up(v):
    e = lax.bitcast_convert_type(lax.shift_left(v, 16), jnp.float32)
    o = lax.bitcast_convert_type(
        jnp.bitwise_and(v, jnp.int32(-65536)), jnp.float32)
    return e, o


def _sc_body(idxp, dstp, tall, w2b, b2v, entp, ball, ipp,
             hall, hsp,
             agg, sbig, ubuf, ibuf, dbuf, ipb, w2m, b2m, bvm,
             isem, gsem):
    c = lax.axis_index("c")
    s = lax.axis_index("s")
    cf = lax.convert_element_type(c, jnp.float32)
    s_negsign = 2.0 * cf - 1.0

    pltpu.sync_copy(w2b, w2m)
    pltpu.sync_copy(b2v, b2m)
    pltpu.sync_copy(ball.at[c], bvm)
    w2u = []
    for j2 in range(4):
        sl = pl.ds(16 * j2, 16)
        w2u.extend(_up(w2m[sl]))
    bvr = [bvm[pl.ds(16 * j, 16)] for j in range(8)]
    b2s = b2m[pl.ds(0, 16)][0]
    lanes = lax.broadcasted_iota(jnp.int32, (16,), 0)

    def _zrow(e, _):
        for j in range(8):
            ubuf[e, pl.ds(16 * j, 16)] = jnp.zeros((16,), jnp.float32)
        return 0
    lax.fori_loop(0, K, _zrow, 0)

    def _zinit(i, _):
        pltpu.sync_copy(ubuf, agg.at[pl.ds(s * NROWS + i * K, K)])
        return 0
    lax.fori_loop(0, NROWS // K, _zinit, 0)
    plsc.subcore_barrier()

    q0 = s * CH

    def _issue_idx(ch):
        slot = lax.rem(ch, 3)
        pltpu.async_copy(idxp.at[c, q0 + ch], ibuf.at[slot], isem.at[slot])
        pltpu.async_copy(dstp.at[q0 + ch], dbuf.at[slot], isem.at[slot])

    def _wait_idx(ch):
        slot = lax.rem(ch, 3)
        pltpu.make_async_copy(idxp.at[0, 0], ibuf.at[0], isem.at[slot]).wait()
        pltpu.make_async_copy(dstp.at[0], dbuf.at[0], isem.at[slot]).wait()

    def _issue_gather(ch, par):
        slot = lax.rem(ch, 3)
        pltpu.async_copy(tall.at[ibuf.at[slot]], sbig.at[pl.ds(par * GK, GK)],
                         gsem.at[par])

    def _wait_gather(par):
        pltpu.make_async_copy(tall.at[pl.ds(0, GK)],
                              sbig.at[pl.ds(0, GK)], gsem.at[par]).wait()

    _issue_idx(0)
    _issue_idx(1)
    _wait_idx(0)
    _issue_gather(0, 0)

    def _chunk(g, _):
        p = lax.rem(g, 2)
        slot = lax.rem(g, 3)

        @pl.when(g + 2 < CH)
        def _():
            _issue_idx(g + 2)

        @pl.when(g + 1 < CH)
        def _():
            _wait_idx(g + 1)
            _issue_gather(g + 1, 1 - p)

        _wait_gather(p)
        p96 = p * GK

        def _edge_one(e):
            srow = p96 + e
            rrow = p96 + K + e
            trow = p96 + 2 * K + e
            acc = jnp.zeros((16,), jnp.float32)
            for j2 in range(4):
                sl = pl.ds(16 * j2, 16)
                s0, s1 = _up(sbig[srow, sl])
                r0, r1 = _up(sbig[rrow, sl])
                t0, t1 = _up(sbig[trow, sl])
                a0 = s0 + r0 + t0
                a1 = s1 + r1 + t1
                acc = (acc + jnp.maximum(a0, 0.0) * w2u[2 * j2]
                       + jnp.maximum(a1, 0.0) * w2u[2 * j2 + 1])
            for sh in (1, 2, 4, 8):
                acc = acc + lax.gather(
                    acc, (lanes ^ sh)[:, None],
                    dimension_numbers=lax.GatherDimensionNumbers(
                        offset_dims=(), collapsed_slice_dims=(0,),
                        start_index_map=(0,)),
                    slice_sizes=(1,),
                    mode=lax.GatherScatterMode.PROMISE_IN_BOUNDS)
            alpha = acc + b2s
            ex = jnp.exp(alpha * s_negsign)
            wv = 1.0 / (1.0 + ex)
            for j2 in range(4):
                sl = pl.ds(64 + 16 * j2, 16)
                s0, s1 = _up(sbig[srow, sl])
                r0, r1 = _up(sbig[rrow, sl])
                t0, t1 = _up(sbig[trow, sl])
                ubuf[e, pl.ds(32 * j2, 16)] = (s0 + r0 + t0) * wv
                ubuf[e, pl.ds(32 * j2 + 16, 16)] = (s1 + r1 + t1) * wv

        def _edge(i, _):
            _edge_one(2 * i)
            _edge_one(2 * i + 1)
            return 0
        lax.fori_loop(0, K // 2, _edge, 0)
        pltpu.sync_copy(ubuf, agg.at[dbuf.at[slot]], add=True)
        return 0
    lax.fori_loop(0, CH, _chunk, 0)
    plsc.subcore_barrier()

    hbase = c * NPAD

    def _node(i, _):
        r0 = s * NROWS + i * K
        pltpu.sync_copy(agg.at[pl.ds(r0, K)], ubuf)
        pltpu.sync_copy(entp.at[pl.ds(hbase + r0, K)], sbig.at[pl.ds(0, K)])

        def _row(r, _):
            for j in range(8):
                sl = pl.ds(16 * j, 16)
                ev = lax.bitcast_convert_type(sbig[r, sl], jnp.float32)
                hv = jnp.maximum(ubuf[r, sl] + bvr[j], 0.0) + ev
                ubuf[r, sl] = hv
            return 0
        lax.fori_loop(0, K, _row, 0)
        pltpu.sync_copy(ubuf, hall.at[pl.ds(hbase + r0, K)])

        @pl.when(c == 1)
        def _():
            pltpu.sync_copy(ipp.at[pl.ds(r0, K)], ipb)
            pltpu.sync_copy(ubuf, hsp.at[ipb])
        return 0
    lax.fori_loop(0, NROWS // K, _node, 0)


def _sc_edge(idxp, dstp, tall, w2b, b2v, entp, ball, ipp):
    mesh = plsc.VectorSubcoreMesh(
        core_axis_name="c", subcore_axis_name="s", num_cores=NC,
        num_subcores=NS)
    f = pl.kernel(
        _sc_body,
        out_type=[
            jax.ShapeDtypeStruct((2 * NPAD, D), jnp.float32),
            jax.ShapeDtypeStruct((N + 16, D), jnp.float32),
        ],
        mesh=mesh,
        scratch_types=[
            pltpu.VMEM_SHARED((NPAD, D), jnp.float32),
            pltpu.VMEM((2 * GK, D), jnp.int32),
            pltpu.VMEM((K, D), jnp.float32),
            pltpu.VMEM((3, GK), jnp.int32),
            pltpu.VMEM((3, K), jnp.int32),
            pltpu.VMEM((K,), jnp.int32),
            pltpu.VMEM((D // 2,), jnp.int32),
            pltpu.VMEM((16,), jnp.float32),
            pltpu.VMEM((D,), jnp.float32),
            pltpu.SemaphoreType.DMA((3,)),
            pltpu.SemaphoreType.DMA((2,)),
        ],
    )
    return f(idxp, dstp, tall, w2b, b2v, entp, ball, ipp)


def _heads_body(hb, sb, pb, wc, ws_, wd, bc2, bs2, bd2, pc, ps, pd):
    hcb = hb[...]
    hsb = sb[...]
    hdb = hcb + pb[...]
    pc[...] = jnp.dot(hcb, wc[...], preferred_element_type=jnp.float32) + bc2[...]
    ps[...] = jnp.dot(hsb, ws_[...], preferred_element_type=jnp.float32) + bs2[...]
    pd[...] = jnp.dot(hdb, wd[...], preferred_element_type=jnp.float32) + bd2[...]


def _heads(hb, sb, pb, wc, ws_, wd, bc2, bs2, bd2):
    BM, BN = 512, 2048
    grid = (pl.cdiv(N, BN), pl.cdiv(N, BM))
    hspec = pl.BlockSpec((BM, D), lambda ni, mj: (mj, 0))
    wspec = pl.BlockSpec((D, BN), lambda ni, mj: (0, ni))
    bspec = pl.BlockSpec((1, BN), lambda ni, mj: (0, ni))
    ospec = pl.BlockSpec((BM, BN), lambda ni, mj: (mj, ni))
    oshape = jax.ShapeDtypeStruct((N, N), jnp.float32)
    return pl.pallas_call(
        _heads_body,
        grid=grid,
        in_specs=[hspec, hspec, hspec, wspec, wspec, wspec, bspec, bspec,
                  bspec],
        out_specs=[ospec, ospec, ospec],
        out_shape=[oshape, oshape, oshape],
    )(hb, sb, pb, wc, ws_, wd, bc2, bs2, bd2)


def kernel(edge_index, edge_type, edge_time, query_rel, entity_emb_c,
           rel_emb_c, time_emb_c, Wc, bc, entity_emb_s, rel_emb_s, time_emb_s,
           Ws, bs, W1, b1, W2, b2, Wpc, bpc, Wps, bps, Wpdo, bpdo):
    f32 = jnp.float32
    i32 = jnp.int32
    src = jnp.asarray(edge_index[0], i32)
    dst = jnp.asarray(edge_index[1], i32)
    typ = jnp.asarray(edge_type, i32)
    tim = jnp.asarray(edge_time, i32)
    pad = EP - E
    srcp = jnp.concatenate([src, jnp.zeros((pad,), i32)]).reshape(NCHT, K)
    dstp = jnp.concatenate([dst, jnp.full((pad,), N, i32)]).reshape(NCHT, K)
    typp = jnp.concatenate([typ, jnp.zeros((pad,), i32)]).reshape(NCHT, K)
    timp = jnp.concatenate([tim, jnp.zeros((pad,), i32)]).reshape(NCHT, K)
    ga_c = jnp.stack(
        [srcp, 2 * N + typp, 2 * N + 2 * R + timp], 1).reshape(NCHT, GK)
    ga_s = jnp.stack(
        [N + srcp, 2 * N + R + typp, 2 * N + 2 * R + TPAD + timp],
        1).reshape(NCHT, GK)
    idxp = jnp.stack([ga_c, ga_s], 0)

    W1a, W1b, W1c, W1d = W1[:D], W1[D:2 * D], W1[2 * D:3 * D], W1[3 * D:]
    rq = lax.dynamic_slice(rel_emb_c, (query_rel, 0), (1, D))
    c08 = _mm(jnp.broadcast_to(rq, (8, D)), W1c)
    c0v = c08[0] + b1

    TEc = _mm(entity_emb_c, jnp.concatenate([W1a, Wc], 1))
    TEs = _mm(jnp.concatenate([entity_emb_c, entity_emb_s], 1),
              _blockdiag(W1a, Ws))
    TRc = _mm(rel_emb_c, jnp.concatenate([W1b, Wc], 1))
    TRs = _mm(jnp.concatenate([rel_emb_c, rel_emb_s], 1), _blockdiag(W1b, Ws))
    tcp = jnp.concatenate([time_emb_c, jnp.zeros((TPAD - 365, D), f32)], 0)
    tsp = jnp.concatenate([time_emb_s, jnp.zeros((TPAD - 365, D), f32)], 0)
    ones = jnp.ones((TPAD, 1), f32)
    z127 = jnp.zeros((TPAD, 127), f32)
    zrow = jnp.zeros((127, 2 * D), f32)
    c0row = jnp.concatenate([c0v, jnp.zeros((D,), f32)]).reshape(1, 2 * D)
    Wtc = jnp.concatenate(
        [jnp.concatenate([W1d, Wc], 1), c0row, zrow], 0)
    TTc = _mm(jnp.concatenate([tcp, ones, z127], 1), Wtc)
    Wts = jnp.concatenate([_blockdiag(W1d, Ws), c0row, zrow], 0)
    TTs = _mm(jnp.concatenate([tcp, tsp, ones, z127], 1), Wts)
    tall = _ileave(jnp.concatenate([TEc, TEs, TRc, TRs, TTc, TTs], 0))

    w2b = _ileave(W2[:, 0].reshape(1, D))[0]
    b2v = jnp.full((16,), b2[0], f32)
    zpad = jnp.zeros((NPAD - N, D), f32)
    entp = lax.bitcast_convert_type(
        jnp.concatenate([entity_emb_c, zpad, entity_emb_s, zpad], 0),
        jnp.int32)
    ball = jnp.stack([bc, bs], 0)

    perm = jax.random.permutation(jax.random.key(42), N)
    inv = jnp.zeros((N,), i32).at[perm].set(jnp.arange(N, dtype=i32))
    ipp = jnp.concatenate([inv, jnp.full((NPAD - N,), N, i32)])

    hall, hsp = _sc_edge(idxp, dstp, tall, w2b, b2v, entp, ball, ipp)
    hc = hall[:N]
    hs = hall[NPAD:NPAD + N]
    hs_perm = hsp[:N]

    bf16 = jnp.bfloat16
    pc, ps, pdo = _heads(
        hc.astype(bf16), hs.astype(bf16), hs_perm.astype(bf16),
        Wpc.astype(bf16), Wps.astype(bf16), Wpdo.astype(bf16),
        bpc.reshape(1, N), bps.reshape(1, N), bpdo.reshape(1, N))
    return (pc, ps, pdo, hc, hs)


# speedup vs baseline: 1.0018x; 1.0018x over previous
import functools

import jax
import jax.numpy as jnp
from jax import lax
from jax.experimental import pallas as pl
from jax.experimental.pallas import tpu as pltpu
from jax.experimental.pallas import tpu_sc as plsc

N = 10000
E = 160000
D = 128
R = 200
TPAD = 368
NC, NS = 2, 16
K = 40
CH = 250
EPT = CH * K
EP = EPT * NS
NCHT = NS * CH
GK = 3 * K
NPAD = 10240
NROWS = NPAD // NS
TROWS = 2 * N + 2 * R + 2 * TPAD


def _mm_body(xr, wr, outr):
    outr[...] = jnp.dot(xr[...], wr[...], preferred_element_type=jnp.float32)


def _mm(x, w):
    m, k = x.shape
    n = w.shape[1]
    bm = min(m, 512)
    return pl.pallas_call(
        _mm_body,
        grid=(pl.cdiv(m, bm),),
        in_specs=[
            pl.BlockSpec((bm, k), lambda i: (i, 0)),
            pl.BlockSpec((k, n), lambda i: (0, 0)),
        ],
        out_specs=pl.BlockSpec((bm, n), lambda i: (i, 0)),
        out_shape=jax.ShapeDtypeStruct((m, n), jnp.float32),
    )(x, w)


def _blockdiag(a, b):
    z = jnp.zeros((a.shape[0], b.shape[1]), jnp.float32)
    z2 = jnp.zeros((b.shape[0], a.shape[1]), jnp.float32)
    return jnp.concatenate(
        [jnp.concatenate([a, z], 1), jnp.concatenate([z2, b], 1)], 0)


def _ileave(x):
    r, c = x.shape
    y = x.reshape(r, c // 32, 2, 16).swapaxes(2, 3).reshape(r, c // 2, 2)
    return lax.bitcast_convert_type(y.astype(jnp.bfloat16), jnp.int32)


def _up(v):
    e = lax.bitcast_convert_type(lax.shift_left(v, 16), jnp.float32)
    o = lax.bitcast_convert_type(
        jnp.bitwise_and(v, jnp.int32(-65536)), jnp.float32)
    return e, o


def _sc_body(idxp, dstp, tall, w2b, b2v, entp, ball, ipp,
             hall, hsp,
             agg, sbig, ubuf, ibuf, dbuf, ipb, w2m, b2m, bvm,
             isem, gsem):
    c = lax.axis_index("c")
    s = lax.axis_index("s")
    cf = lax.convert_element_type(c, jnp.float32)
    s_negsign = 2.0 * cf - 1.0

    pltpu.sync_copy(w2b, w2m)
    pltpu.sync_copy(b2v, b2m)
    pltpu.sync_copy(ball.at[c], bvm)
    w2u = []
    for j2 in range(4):
        sl = pl.ds(16 * j2, 16)
        w2u.extend(_up(w2m[sl]))
    bvr = [bvm[pl.ds(16 * j, 16)] for j in range(8)]
    b2s = b2m[pl.ds(0, 16)][0]
    lanes = lax.broadcasted_iota(jnp.int32, (16,), 0)

    def _zrow(e, _):
        for j in range(8):
            ubuf[e, pl.ds(16 * j, 16)] = jnp.zeros((16,), jnp.float32)
        return 0
    lax.fori_loop(0, K, _zrow, 0)

    def _zinit(i, _):
        pltpu.sync_copy(ubuf, agg.at[pl.ds(s * NROWS + i * K, K)])
        return 0
    lax.fori_loop(0, NROWS // K, _zinit, 0)
    plsc.subcore_barrier()

    q0 = s * CH

    def _issue_idx(ch):
        slot = lax.rem(ch, 3)
        pltpu.async_copy(idxp.at[c, q0 + ch], ibuf.at[slot], isem.at[slot])
        pltpu.async_copy(dstp.at[q0 + ch], dbuf.at[slot], isem.at[slot])

    def _wait_idx(ch):
        slot = lax.rem(ch, 3)
        pltpu.make_async_copy(idxp.at[0, 0], ibuf.at[0], isem.at[slot]).wait()
        pltpu.make_async_copy(dstp.at[0], dbuf.at[0], isem.at[slot]).wait()

    def _issue_gather(ch, par):
        slot = lax.rem(ch, 3)
        pltpu.async_copy(tall.at[ibuf.at[slot]], sbig.at[pl.ds(par * GK, GK)],
                         gsem.at[par])

    def _wait_gather(par):
        pltpu.make_async_copy(tall.at[pl.ds(0, GK)],
                              sbig.at[pl.ds(0, GK)], gsem.at[par]).wait()

    _issue_idx(0)
    _issue_idx(1)
    _wait_idx(0)
    _issue_gather(0, 0)

    def _chunk(g, _):
        p = lax.rem(g, 2)
        slot = lax.rem(g, 3)

        @pl.when(g + 2 < CH)
        def _():
            _issue_idx(g + 2)

        @pl.when(g + 1 < CH)
        def _():
            _wait_idx(g + 1)
            _issue_gather(g + 1, 1 - p)

        _wait_gather(p)
        p96 = p * GK

        def _edge_one(e):
            srow = p96 + e
            rrow = p96 + K + e
            trow = p96 + 2 * K + e
            acc = jnp.zeros((16,), jnp.float32)
            for j2 in range(4):
                sl = pl.ds(16 * j2, 16)
                s0, s1 = _up(sbig[srow, sl])
                r0, r1 = _up(sbig[rrow, sl])
                t0, t1 = _up(sbig[trow, sl])
                a0 = s0 + r0 + t0
                a1 = s1 + r1 + t1
                acc = (acc + jnp.maximum(a0, 0.0) * w2u[2 * j2]
                       + jnp.maximum(a1, 0.0) * w2u[2 * j2 + 1])
            for sh in (1, 2, 4, 8):
                acc = acc + lax.gather(
                    acc, (lanes ^ sh)[:, None],
                    dimension_numbers=lax.GatherDimensionNumbers(
                        offset_dims=(), collapsed_slice_dims=(0,),
                        start_index_map=(0,)),
                    slice_sizes=(1,),
                    mode=lax.GatherScatterMode.PROMISE_IN_BOUNDS)
            alpha = acc + b2s
            ex = jnp.exp(alpha * s_negsign)
            wv = 1.0 / (1.0 + ex)
            for j2 in range(4):
                sl = pl.ds(64 + 16 * j2, 16)
                s0, s1 = _up(sbig[srow, sl])
                r0, r1 = _up(sbig[rrow, sl])
                t0, t1 = _up(sbig[trow, sl])
                ubuf[e, pl.ds(32 * j2, 16)] = (s0 + r0 + t0) * wv
                ubuf[e, pl.ds(32 * j2 + 16, 16)] = (s1 + r1 + t1) * wv

        def _edge(i, _):
            for u in range(4):
                _edge_one(4 * i + u)
            return 0
        lax.fori_loop(0, K // 4, _edge, 0)
        pltpu.sync_copy(ubuf, agg.at[dbuf.at[slot]], add=True)
        return 0
    lax.fori_loop(0, CH, _chunk, 0)
    plsc.subcore_barrier()

    hbase = c * NPAD

    def _node(i, _):
        r0 = s * NROWS + i * K
        pltpu.sync_copy(agg.at[pl.ds(r0, K)], ubuf)
        pltpu.sync_copy(entp.at[pl.ds(hbase + r0, K)], sbig.at[pl.ds(0, K)])

        def _row(r, _):
            for j in range(8):
                sl = pl.ds(16 * j, 16)
                ev = lax.bitcast_convert_type(sbig[r, sl], jnp.float32)
                hv = jnp.maximum(ubuf[r, sl] + bvr[j], 0.0) + ev
                ubuf[r, sl] = hv
            return 0
        lax.fori_loop(0, K, _row, 0)
        pltpu.sync_copy(ubuf, hall.at[pl.ds(hbase + r0, K)])

        @pl.when(c == 1)
        def _():
            pltpu.sync_copy(ipp.at[pl.ds(r0, K)], ipb)
            pltpu.sync_copy(ubuf, hsp.at[ipb])
        return 0
    lax.fori_loop(0, NROWS // K, _node, 0)


def _sc_edge(idxp, dstp, tall, w2b, b2v, entp, ball, ipp):
    mesh = plsc.VectorSubcoreMesh(
        core_axis_name="c", subcore_axis_name="s", num_cores=NC,
        num_subcores=NS)
    f = pl.kernel(
        _sc_body,
        out_type=[
            jax.ShapeDtypeStruct((2 * NPAD, D), jnp.float32),
            jax.ShapeDtypeStruct((N + 16, D), jnp.float32),
        ],
        mesh=mesh,
        scratch_types=[
            pltpu.VMEM_SHARED((NPAD, D), jnp.float32),
            pltpu.VMEM((2 * GK, D), jnp.int32),
            pltpu.VMEM((K, D), jnp.float32),
            pltpu.VMEM((3, GK), jnp.int32),
            pltpu.VMEM((3, K), jnp.int32),
            pltpu.VMEM((K,), jnp.int32),
            pltpu.VMEM((D // 2,), jnp.int32),
            pltpu.VMEM((16,), jnp.float32),
            pltpu.VMEM((D,), jnp.float32),
            pltpu.SemaphoreType.DMA((3,)),
            pltpu.SemaphoreType.DMA((2,)),
        ],
    )
    return f(idxp, dstp, tall, w2b, b2v, entp, ball, ipp)


def _heads_body(hb, sb, pb, wc, ws_, wd, bc2, bs2, bd2, pc, ps, pd):
    hcb = hb[...]
    hsb = sb[...]
    hdb = hcb + pb[...]
    pc[...] = jnp.dot(hcb, wc[...], preferred_element_type=jnp.float32) + bc2[...]
    ps[...] = jnp.dot(hsb, ws_[...], preferred_element_type=jnp.float32) + bs2[...]
    pd[...] = jnp.dot(hdb, wd[...], preferred_element_type=jnp.float32) + bd2[...]


def _heads(hb, sb, pb, wc, ws_, wd, bc2, bs2, bd2):
    BM, BN = 512, 2048
    grid = (pl.cdiv(N, BN), pl.cdiv(N, BM))
    hspec = pl.BlockSpec((BM, D), lambda ni, mj: (mj, 0))
    wspec = pl.BlockSpec((D, BN), lambda ni, mj: (0, ni))
    bspec = pl.BlockSpec((1, BN), lambda ni, mj: (0, ni))
    ospec = pl.BlockSpec((BM, BN), lambda ni, mj: (mj, ni))
    oshape = jax.ShapeDtypeStruct((N, N), jnp.float32)
    return pl.pallas_call(
        _heads_body,
        grid=grid,
        in_specs=[hspec, hspec, hspec, wspec, wspec, wspec, bspec, bspec,
                  bspec],
        out_specs=[ospec, ospec, ospec],
        out_shape=[oshape, oshape, oshape],
    )(hb, sb, pb, wc, ws_, wd, bc2, bs2, bd2)


def kernel(edge_index, edge_type, edge_time, query_rel, entity_emb_c,
           rel_emb_c, time_emb_c, Wc, bc, entity_emb_s, rel_emb_s, time_emb_s,
           Ws, bs, W1, b1, W2, b2, Wpc, bpc, Wps, bps, Wpdo, bpdo):
    f32 = jnp.float32
    i32 = jnp.int32
    src = jnp.asarray(edge_index[0], i32)
    dst = jnp.asarray(edge_index[1], i32)
    typ = jnp.asarray(edge_type, i32)
    tim = jnp.asarray(edge_time, i32)
    pad = EP - E
    srcp = jnp.concatenate([src, jnp.zeros((pad,), i32)]).reshape(NCHT, K)
    dstp = jnp.concatenate([dst, jnp.full((pad,), N, i32)]).reshape(NCHT, K)
    typp = jnp.concatenate([typ, jnp.zeros((pad,), i32)]).reshape(NCHT, K)
    timp = jnp.concatenate([tim, jnp.zeros((pad,), i32)]).reshape(NCHT, K)
    ga_c = jnp.stack(
        [srcp, 2 * N + typp, 2 * N + 2 * R + timp], 1).reshape(NCHT, GK)
    ga_s = jnp.stack(
        [N + srcp, 2 * N + R + typp, 2 * N + 2 * R + TPAD + timp],
        1).reshape(NCHT, GK)
    idxp = jnp.stack([ga_c, ga_s], 0)

    W1a, W1b, W1c, W1d = W1[:D], W1[D:2 * D], W1[2 * D:3 * D], W1[3 * D:]
    rq = lax.dynamic_slice(rel_emb_c, (query_rel, 0), (1, D))
    c08 = _mm(jnp.broadcast_to(rq, (8, D)), W1c)
    c0v = c08[0] + b1

    TEc = _mm(entity_emb_c, jnp.concatenate([W1a, Wc], 1))
    TEs = _mm(jnp.concatenate([entity_emb_c, entity_emb_s], 1),
              _blockdiag(W1a, Ws))
    TRc = _mm(rel_emb_c, jnp.concatenate([W1b, Wc], 1))
    TRs = _mm(jnp.concatenate([rel_emb_c, rel_emb_s], 1), _blockdiag(W1b, Ws))
    tcp = jnp.concatenate([time_emb_c, jnp.zeros((TPAD - 365, D), f32)], 0)
    tsp = jnp.concatenate([time_emb_s, jnp.zeros((TPAD - 365, D), f32)], 0)
    ones = jnp.ones((TPAD, 1), f32)
    z127 = jnp.zeros((TPAD, 127), f32)
    zrow = jnp.zeros((127, 2 * D), f32)
    c0row = jnp.concatenate([c0v, jnp.zeros((D,), f32)]).reshape(1, 2 * D)
    Wtc = jnp.concatenate(
        [jnp.concatenate([W1d, Wc], 1), c0row, zrow], 0)
    TTc = _mm(jnp.concatenate([tcp, ones, z127], 1), Wtc)
    Wts = jnp.concatenate([_blockdiag(W1d, Ws), c0row, zrow], 0)
    TTs = _mm(jnp.concatenate([tcp, tsp, ones, z127], 1), Wts)
    tall = _ileave(jnp.concatenate([TEc, TEs, TRc, TRs, TTc, TTs], 0))

    w2b = _ileave(W2[:, 0].reshape(1, D))[0]
    b2v = jnp.full((16,), b2[0], f32)
    zpad = jnp.zeros((NPAD - N, D), f32)
    entp = lax.bitcast_convert_type(
        jnp.concatenate([entity_emb_c, zpad, entity_emb_s, zpad], 0),
        jnp.int32)
    ball = jnp.stack([bc, bs], 0)

    perm = jax.random.permutation(jax.random.key(42), N)
    inv = jnp.zeros((N,), i32).at[perm].set(jnp.arange(N, dtype=i32))
    ipp = jnp.concatenate([inv, jnp.full((NPAD - N,), N, i32)])

    hall, hsp = _sc_edge(idxp, dstp, tall, w2b, b2v, entp, ball, ipp)
    hc = hall[:N]
    hs = hall[NPAD:NPAD + N]
    hs_perm = hsp[:N]

    bf16 = jnp.bfloat16
    pc, ps, pdo = _heads(
        hc.astype(bf16), hs.astype(bf16), hs_perm.astype(bf16),
        Wpc.astype(bf16), Wps.astype(bf16), Wpdo.astype(bf16),
        bpc.reshape(1, N), bps.reshape(1, N), bpdo.reshape(1, N))
    return (pc, ps, pdo, hc, hs)


# speedup vs baseline: 1.0042x; 1.0024x over previous
import jax
import jax.numpy as jnp
from jax import lax
from jax.experimental import pallas as pl
from jax.experimental.pallas import tpu as pltpu
from jax.experimental.pallas import tpu_sc as plsc

N = 10000
E = 160000
D = 128
R = 200
TPAD = 368
NC, NS = 2, 16
K = 40
CH = 250
EPT = CH * K
EP = EPT * NS
NCHT = NS * CH
GK = 3 * K
NPAD = 10240
NROWS = NPAD // NS
TROWS = 2 * N + 2 * R + 2 * TPAD


def _mm_body(xr, wr, outr):
    outr[...] = jnp.dot(xr[...], wr[...], preferred_element_type=jnp.float32)


def _mm(x, w):
    m, k = x.shape
    n = w.shape[1]
    bm = min(m, 512)
    return pl.pallas_call(
        _mm_body,
        grid=(pl.cdiv(m, bm),),
        in_specs=[
            pl.BlockSpec((bm, k), lambda i: (i, 0)),
            pl.BlockSpec((k, n), lambda i: (0, 0)),
        ],
        out_specs=pl.BlockSpec((bm, n), lambda i: (i, 0)),
        out_shape=jax.ShapeDtypeStruct((m, n), jnp.float32),
    )(x, w)


def _blockdiag(a, b):
    z = jnp.zeros((a.shape[0], b.shape[1]), jnp.float32)
    z2 = jnp.zeros((b.shape[0], a.shape[1]), jnp.float32)
    return jnp.concatenate(
        [jnp.concatenate([a, z], 1), jnp.concatenate([z2, b], 1)], 0)


def _ileave(x):
    r, c = x.shape
    y = x.reshape(r, c // 32, 2, 16).swapaxes(2, 3).reshape(r, c // 2, 2)
    return lax.bitcast_convert_type(y.astype(jnp.bfloat16), jnp.int32)


def _up(v):
    e = lax.bitcast_convert_type(lax.shift_left(v, 16), jnp.float32)
    o = lax.bitcast_convert_type(
        jnp.bitwise_and(v, jnp.int32(-65536)), jnp.float32)
    return e, o


def _sc_body(idxp, dstp, tall, w2b, b2v, entp, ball, ipp,
             hall, hsp,
             agg, sbig, ubuf, ibuf, dbuf, ipb, w2m, b2m, bvm,
             isem, gsem):
    c = lax.axis_index("c")
    s = lax.axis_index("s")
    cf = lax.convert_element_type(c, jnp.float32)
    s_negsign = 2.0 * cf - 1.0

    pltpu.sync_copy(w2b, w2m)
    pltpu.sync_copy(b2v, b2m)
    pltpu.sync_copy(ball.at[c], bvm)
    w2u = []
    for j2 in range(4):
        sl = pl.ds(16 * j2, 16)
        w2u.extend(_up(w2m[sl]))
    bvr = [bvm[pl.ds(16 * j, 16)] for j in range(8)]
    b2s = b2m[pl.ds(0, 16)][0]
    lanes = lax.broadcasted_iota(jnp.int32, (16,), 0)

    def _zrow(e, _):
        for j in range(8):
            ubuf[e, pl.ds(16 * j, 16)] = jnp.zeros((16,), jnp.float32)
        return 0
    lax.fori_loop(0, K, _zrow, 0)

    def _zinit(i, _):
        pltpu.sync_copy(ubuf, agg.at[pl.ds(s * NROWS + i * K, K)])
        return 0
    lax.fori_loop(0, NROWS // K, _zinit, 0)
    plsc.subcore_barrier()

    q0 = s * CH

    def _issue_idx(ch):
        slot = lax.rem(ch, 3)
        pltpu.async_copy(idxp.at[c, q0 + ch], ibuf.at[slot], isem.at[slot])
        pltpu.async_copy(dstp.at[q0 + ch], dbuf.at[slot], isem.at[slot])

    def _wait_idx(ch):
        slot = lax.rem(ch, 3)
        pltpu.make_async_copy(idxp.at[0, 0], ibuf.at[0], isem.at[slot]).wait()
        pltpu.make_async_copy(dstp.at[0], dbuf.at[0], isem.at[slot]).wait()

    def _issue_gather(ch, par):
        slot = lax.rem(ch, 3)
        pltpu.async_copy(tall.at[ibuf.at[slot]], sbig.at[pl.ds(par * GK, GK)],
                         gsem.at[par])

    def _wait_gather(par):
        pltpu.make_async_copy(tall.at[pl.ds(0, GK)],
                              sbig.at[pl.ds(0, GK)], gsem.at[par]).wait()

    _issue_idx(0)
    _issue_idx(1)
    _wait_idx(0)
    _issue_gather(0, 0)

    def _chunk(g, _):
        p = lax.rem(g, 2)
        slot = lax.rem(g, 3)

        @pl.when(g + 2 < CH)
        def _():
            _issue_idx(g + 2)

        @pl.when(g + 1 < CH)
        def _():
            _wait_idx(g + 1)
            _issue_gather(g + 1, 1 - p)

        _wait_gather(p)
        p96 = p * GK

        def _edge_one(e):
            srow = p96 + e
            rrow = p96 + K + e
            trow = p96 + 2 * K + e
            acc = jnp.zeros((16,), jnp.float32)
            for j2 in range(4):
                sl = pl.ds(16 * j2, 16)
                s0, s1 = _up(sbig[srow, sl])
                r0, r1 = _up(sbig[rrow, sl])
                t0, t1 = _up(sbig[trow, sl])
                a0 = s0 + r0 + t0
                a1 = s1 + r1 + t1
                acc = (acc + jnp.maximum(a0, 0.0) * w2u[2 * j2]
                       + jnp.maximum(a1, 0.0) * w2u[2 * j2 + 1])
            for sh in (1, 2, 4, 8):
                acc = acc + lax.gather(
                    acc, (lanes ^ sh)[:, None],
                    dimension_numbers=lax.GatherDimensionNumbers(
                        offset_dims=(), collapsed_slice_dims=(0,),
                        start_index_map=(0,)),
                    slice_sizes=(1,),
                    mode=lax.GatherScatterMode.PROMISE_IN_BOUNDS)
            alpha = acc + b2s
            ex = jnp.exp(alpha * s_negsign)
            wv = 1.0 / (1.0 + ex)
            for j2 in range(4):
                sl = pl.ds(64 + 16 * j2, 16)
                s0, s1 = _up(sbig[srow, sl])
                r0, r1 = _up(sbig[rrow, sl])
                t0, t1 = _up(sbig[trow, sl])
                ubuf[e, pl.ds(32 * j2, 16)] = (s0 + r0 + t0) * wv
                ubuf[e, pl.ds(32 * j2 + 16, 16)] = (s1 + r1 + t1) * wv

        def _edge(i, _):
            for u in range(4):
                _edge_one(4 * i + u)
            return 0
        lax.fori_loop(0, K // 4, _edge, 0)
        pltpu.sync_copy(ubuf, agg.at[dbuf.at[slot]], add=True)
        return 0
    lax.fori_loop(0, CH, _chunk, 0)
    plsc.subcore_barrier()

    hbase = c * NPAD

    def _node(i, _):
        r0 = s * NROWS + i * K
        pltpu.sync_copy(agg.at[pl.ds(r0, K)], ubuf)
        pltpu.sync_copy(entp.at[pl.ds(hbase + r0, K)], sbig.at[pl.ds(0, K)])

        def _row(r, _):
            for j in range(8):
                sl = pl.ds(16 * j, 16)
                ev = lax.bitcast_convert_type(sbig[r, sl], jnp.float32)
                hv = jnp.maximum(ubuf[r, sl] + bvr[j], 0.0) + ev
                ubuf[r, sl] = hv
            return 0
        lax.fori_loop(0, K, _row, 0)
        pltpu.sync_copy(ubuf, hall.at[pl.ds(hbase + r0, K)])

        @pl.when(c == 1)
        def _():
            pltpu.sync_copy(ipp.at[pl.ds(r0, K)], ipb)
            pltpu.sync_copy(ubuf, hsp.at[ipb])
        return 0
    lax.fori_loop(0, NROWS // K, _node, 0)


def _sc_edge(idxp, dstp, tall, w2b, b2v, entp, ball, ipp):
    mesh = plsc.VectorSubcoreMesh(
        core_axis_name="c", subcore_axis_name="s", num_cores=NC,
        num_subcores=NS)
    f = pl.kernel(
        _sc_body,
        out_type=[
            jax.ShapeDtypeStruct((2 * NPAD, D), jnp.float32),
            jax.ShapeDtypeStruct((N + 16, D), jnp.float32),
        ],
        mesh=mesh,
        scratch_types=[
            pltpu.VMEM_SHARED((NPAD, D), jnp.float32),
            pltpu.VMEM((2 * GK, D), jnp.int32),
            pltpu.VMEM((K, D), jnp.float32),
            pltpu.VMEM((3, GK), jnp.int32),
            pltpu.VMEM((3, K), jnp.int32),
            pltpu.VMEM((K,), jnp.int32),
            pltpu.VMEM((D // 2,), jnp.int32),
            pltpu.VMEM((16,), jnp.float32),
            pltpu.VMEM((D,), jnp.float32),
            pltpu.SemaphoreType.DMA((3,)),
            pltpu.SemaphoreType.DMA((2,)),
        ],
    )
    return f(idxp, dstp, tall, w2b, b2v, entp, ball, ipp)


def _heads_body(hb, sb, pb, wc, ws_, wd, bc2, bs2, bd2, pc, ps, pd):
    hcb = hb[...]
    hsb = sb[...]
    hdb = hcb + pb[...]
    pc[...] = jnp.dot(hcb, wc[...], preferred_element_type=jnp.float32) + bc2[...]
    ps[...] = jnp.dot(hsb, ws_[...], preferred_element_type=jnp.float32) + bs2[...]
    pd[...] = jnp.dot(hdb, wd[...], preferred_element_type=jnp.float32) + bd2[...]


def _heads(hb, sb, pb, wc, ws_, wd, bc2, bs2, bd2):
    BM, BN = 512, 2048
    grid = (pl.cdiv(N, BN), pl.cdiv(N, BM))
    hspec = pl.BlockSpec((BM, D), lambda ni, mj: (mj, 0))
    wspec = pl.BlockSpec((D, BN), lambda ni, mj: (0, ni))
    bspec = pl.BlockSpec((1, BN), lambda ni, mj: (0, ni))
    ospec = pl.BlockSpec((BM, BN), lambda ni, mj: (mj, ni))
    oshape = jax.ShapeDtypeStruct((N, N), jnp.float32)
    return pl.pallas_call(
        _heads_body,
        grid=grid,
        in_specs=[hspec, hspec, hspec, wspec, wspec, wspec, bspec, bspec,
                  bspec],
        out_specs=[ospec, ospec, ospec],
        out_shape=[oshape, oshape, oshape],
    )(hb, sb, pb, wc, ws_, wd, bc2, bs2, bd2)


def kernel(edge_index, edge_type, edge_time, query_rel, entity_emb_c,
           rel_emb_c, time_emb_c, Wc, bc, entity_emb_s, rel_emb_s, time_emb_s,
           Ws, bs, W1, b1, W2, b2, Wpc, bpc, Wps, bps, Wpdo, bpdo):
    f32 = jnp.float32
    i32 = jnp.int32
    src = jnp.asarray(edge_index[0], i32)
    dst = jnp.asarray(edge_index[1], i32)
    typ = jnp.asarray(edge_type, i32)
    tim = jnp.asarray(edge_time, i32)
    pad = EP - E
    srcp = jnp.concatenate([src, jnp.zeros((pad,), i32)]).reshape(NCHT, K)
    dstp = jnp.concatenate([dst, jnp.full((pad,), N, i32)]).reshape(NCHT, K)
    typp = jnp.concatenate([typ, jnp.zeros((pad,), i32)]).reshape(NCHT, K)
    timp = jnp.concatenate([tim, jnp.zeros((pad,), i32)]).reshape(NCHT, K)
    ga_c = jnp.stack(
        [srcp, 2 * N + typp, 2 * N + 2 * R + timp], 1).reshape(NCHT, GK)
    ga_s = jnp.stack(
        [N + srcp, 2 * N + R + typp, 2 * N + 2 * R + TPAD + timp],
        1).reshape(NCHT, GK)
    idxp = jnp.stack([ga_c, ga_s], 0)

    W1a, W1b, W1c, W1d = W1[:D], W1[D:2 * D], W1[2 * D:3 * D], W1[3 * D:]
    rq = lax.dynamic_slice(rel_emb_c, (query_rel, 0), (1, D))
    c08 = _mm(jnp.broadcast_to(rq, (8, D)), W1c)
    c0v = c08[0] + b1

    TEc = _mm(entity_emb_c, jnp.concatenate([W1a, Wc], 1))
    TEs = _mm(jnp.concatenate([entity_emb_c, entity_emb_s], 1),
              _blockdiag(W1a, Ws))
    TRc = _mm(rel_emb_c, jnp.concatenate([W1b, Wc], 1))
    TRs = _mm(jnp.concatenate([rel_emb_c, rel_emb_s], 1), _blockdiag(W1b, Ws))
    tcp = jnp.concatenate([time_emb_c, jnp.zeros((TPAD - 365, D), f32)], 0)
    tsp = jnp.concatenate([time_emb_s, jnp.zeros((TPAD - 365, D), f32)], 0)
    ones = jnp.ones((TPAD, 1), f32)
    z127 = jnp.zeros((TPAD, 127), f32)
    zrow = jnp.zeros((127, 2 * D), f32)
    c0row = jnp.concatenate([c0v, jnp.zeros((D,), f32)]).reshape(1, 2 * D)
    Wtc = jnp.concatenate(
        [jnp.concatenate([W1d, Wc], 1), c0row, zrow], 0)
    TTc = _mm(jnp.concatenate([tcp, ones, z127], 1), Wtc)
    Wts = jnp.concatenate([_blockdiag(W1d, Ws), c0row, zrow], 0)
    TTs = _mm(jnp.concatenate([tcp, tsp, ones, z127], 1), Wts)
    tall = _ileave(jnp.concatenate([TEc, TEs, TRc, TRs, TTc, TTs], 0))

    w2b = _ileave(W2[:, 0].reshape(1, D))[0]
    b2v = jnp.full((16,), b2[0], f32)
    zpad = jnp.zeros((NPAD - N, D), f32)
    entp = lax.bitcast_convert_type(
        jnp.concatenate([entity_emb_c, zpad, entity_emb_s, zpad], 0),
        jnp.int32)
    ball = jnp.stack([bc, bs], 0)

    perm = jax.random.permutation(jax.random.key(42), N)
    inv = jnp.zeros((N,), i32).at[perm].set(jnp.arange(N, dtype=i32))
    ipp = jnp.concatenate([inv, jnp.full((NPAD - N,), N, i32)])

    hall, hsp = _sc_edge(idxp, dstp, tall, w2b, b2v, entp, ball, ipp)
    hc = hall[:N]
    hs = hall[NPAD:NPAD + N]
    hs_perm = hsp[:N]

    bf16 = jnp.bfloat16
    pc, ps, pdo = _heads(
        hc.astype(bf16), hs.astype(bf16), hs_perm.astype(bf16),
        Wpc.astype(bf16), Wps.astype(bf16), Wpdo.astype(bf16),
        bpc.reshape(1, N), bps.reshape(1, N), bpdo.reshape(1, N))
    return (pc, ps, pdo, hc, hs)


# speedup vs baseline: 1.0457x; 1.0413x over previous
import jax
import jax.numpy as jnp
from jax import lax
from jax.experimental import pallas as pl
from jax.experimental.pallas import tpu as pltpu
from jax.experimental.pallas import tpu_sc as plsc

N = 10000
E = 160000
D = 128
R = 200
TPAD = 368
NC, NS = 2, 16
K = 40
CH = 250
EPT = CH * K
EP = EPT * NS
NCHT = NS * CH
GK = 3 * K
NPAD = 10240
NROWS = NPAD // NS
TROWS = 2 * N + 2 * R + 2 * TPAD


def _mm_body(xr, wr, outr):
    outr[...] = jnp.dot(xr[...], wr[...], preferred_element_type=jnp.float32)


def _mm(x, w):
    m, k = x.shape
    n = w.shape[1]
    bm = min(m, 512)
    return pl.pallas_call(
        _mm_body,
        grid=(pl.cdiv(m, bm),),
        in_specs=[
            pl.BlockSpec((bm, k), lambda i: (i, 0)),
            pl.BlockSpec((k, n), lambda i: (0, 0)),
        ],
        out_specs=pl.BlockSpec((bm, n), lambda i: (i, 0)),
        out_shape=jax.ShapeDtypeStruct((m, n), jnp.float32),
    )(x, w)


def _blockdiag(a, b):
    z = jnp.zeros((a.shape[0], b.shape[1]), jnp.float32)
    z2 = jnp.zeros((b.shape[0], a.shape[1]), jnp.float32)
    return jnp.concatenate(
        [jnp.concatenate([a, z], 1), jnp.concatenate([z2, b], 1)], 0)


def _ileave(x):
    r, c = x.shape
    y = x.reshape(r, c // 32, 2, 16).swapaxes(2, 3).reshape(r, c // 2, 2)
    return lax.bitcast_convert_type(y.astype(jnp.bfloat16), jnp.int32)


def _up(v):
    e = lax.bitcast_convert_type(lax.shift_left(v, 16), jnp.float32)
    o = lax.bitcast_convert_type(
        jnp.bitwise_and(v, jnp.int32(-65536)), jnp.float32)
    return e, o


def _sc_body(idxp, dstp, tall, w2b, b2v, entp, ball, ipp,
             hall, hsp,
             agg, sbig, ubuf, ibuf, dbuf, ipb, w2m, b2m, bvm,
             isem, gsem, ssem):
    c = lax.axis_index("c")
    s = lax.axis_index("s")
    cf = lax.convert_element_type(c, jnp.float32)
    s_negsign = 2.0 * cf - 1.0

    pltpu.sync_copy(w2b, w2m)
    pltpu.sync_copy(b2v, b2m)
    pltpu.sync_copy(ball.at[c], bvm)
    w2u = []
    for j2 in range(4):
        sl = pl.ds(16 * j2, 16)
        w2u.extend(_up(w2m[sl]))
    bvr = [bvm[pl.ds(16 * j, 16)] for j in range(8)]
    b2s = b2m[pl.ds(0, 16)][0]
    lanes = lax.broadcasted_iota(jnp.int32, (16,), 0)

    def _zrow(e, _):
        for j in range(8):
            ubuf[e, pl.ds(16 * j, 16)] = jnp.zeros((16,), jnp.float32)
        return 0
    lax.fori_loop(0, K, _zrow, 0)

    def _zinit(i, _):
        pltpu.sync_copy(ubuf.at[pl.ds(0, K)],
                        agg.at[pl.ds(s * NROWS + i * K, K)])
        return 0
    lax.fori_loop(0, NROWS // K, _zinit, 0)
    plsc.subcore_barrier()

    q0 = s * CH

    def _issue_idx(ch):
        slot = lax.rem(ch, 4)
        pltpu.async_copy(idxp.at[c, q0 + ch], ibuf.at[slot], isem.at[slot])
        pltpu.async_copy(dstp.at[q0 + ch], dbuf.at[slot], isem.at[slot])

    def _wait_idx(ch):
        slot = lax.rem(ch, 4)
        pltpu.make_async_copy(idxp.at[0, 0], ibuf.at[0], isem.at[slot]).wait()
        pltpu.make_async_copy(dstp.at[0], dbuf.at[0], isem.at[slot]).wait()

    def _drain_scatter(par):
        pltpu.make_async_copy(hall.at[pl.ds(0, K)],
                              ubuf.at[pl.ds(0, K)], ssem.at[par]).wait()

    def _issue_gather(ch, par):
        slot = lax.rem(ch, 4)
        pltpu.async_copy(tall.at[ibuf.at[slot]], sbig.at[pl.ds(par * GK, GK)],
                         gsem.at[par])

    def _wait_gather(par):
        pltpu.make_async_copy(tall.at[pl.ds(0, GK)],
                              sbig.at[pl.ds(0, GK)], gsem.at[par]).wait()

    _issue_idx(0)
    _issue_idx(1)
    _wait_idx(0)
    _issue_gather(0, 0)

    def _chunk(g, _):
        p = lax.rem(g, 2)
        slot = lax.rem(g, 4)

        @pl.when(g >= 2)
        def _():
            _drain_scatter(p)

        @pl.when(g + 2 < CH)
        def _():
            _issue_idx(g + 2)

        @pl.when(g + 1 < CH)
        def _():
            _wait_idx(g + 1)
            _issue_gather(g + 1, 1 - p)

        _wait_gather(p)
        p96 = p * GK
        pu = p * K

        def _edge_one(e):
            srow = p96 + e
            rrow = p96 + K + e
            trow = p96 + 2 * K + e
            acc = jnp.zeros((16,), jnp.float32)
            for j2 in range(4):
                sl = pl.ds(16 * j2, 16)
                s0, s1 = _up(sbig[srow, sl])
                r0, r1 = _up(sbig[rrow, sl])
                t0, t1 = _up(sbig[trow, sl])
                a0 = s0 + r0 + t0
                a1 = s1 + r1 + t1
                acc = (acc + jnp.maximum(a0, 0.0) * w2u[2 * j2]
                       + jnp.maximum(a1, 0.0) * w2u[2 * j2 + 1])
            for sh in (1, 2, 4, 8):
                acc = acc + lax.gather(
                    acc, (lanes ^ sh)[:, None],
                    dimension_numbers=lax.GatherDimensionNumbers(
                        offset_dims=(), collapsed_slice_dims=(0,),
                        start_index_map=(0,)),
                    slice_sizes=(1,),
                    mode=lax.GatherScatterMode.PROMISE_IN_BOUNDS)
            alpha = acc + b2s
            ex = jnp.exp(alpha * s_negsign)
            wv = 1.0 / (1.0 + ex)
            for j2 in range(4):
                sl = pl.ds(64 + 16 * j2, 16)
                s0, s1 = _up(sbig[srow, sl])
                r0, r1 = _up(sbig[rrow, sl])
                t0, t1 = _up(sbig[trow, sl])
                ubuf[pu + e, pl.ds(32 * j2, 16)] = (s0 + r0 + t0) * wv
                ubuf[pu + e, pl.ds(32 * j2 + 16, 16)] = (s1 + r1 + t1) * wv

        def _edge(i, _):
            for u in range(4):
                _edge_one(4 * i + u)
            return 0
        lax.fori_loop(0, K // 4, _edge, 0)
        pltpu.async_copy(ubuf.at[pl.ds(pu, K)], agg.at[dbuf.at[slot]],
                         ssem.at[p], add=True)
        return 0
    lax.fori_loop(0, CH, _chunk, 0)
    _drain_scatter(CH % 2)
    _drain_scatter(1 - CH % 2)
    plsc.subcore_barrier()

    hbase = c * NPAD

    def _node(i, _):
        r0 = s * NROWS + i * K
        pltpu.sync_copy(agg.at[pl.ds(r0, K)], ubuf.at[pl.ds(0, K)])
        pltpu.sync_copy(entp.at[pl.ds(hbase + r0, K)], sbig.at[pl.ds(0, K)])

        def _row(r, _):
            for j in range(8):
                sl = pl.ds(16 * j, 16)
                ev = lax.bitcast_convert_type(sbig[r, sl], jnp.float32)
                hv = jnp.maximum(ubuf[r, sl] + bvr[j], 0.0) + ev
                ubuf[r, sl] = hv
            return 0
        lax.fori_loop(0, K, _row, 0)
        pltpu.sync_copy(ubuf.at[pl.ds(0, K)], hall.at[pl.ds(hbase + r0, K)])

        @pl.when(c == 1)
        def _():
            pltpu.sync_copy(ipp.at[pl.ds(r0, K)], ipb)
            pltpu.sync_copy(ubuf.at[pl.ds(0, K)], hsp.at[ipb])
        return 0
    lax.fori_loop(0, NROWS // K, _node, 0)


def _sc_edge(idxp, dstp, tall, w2b, b2v, entp, ball, ipp):
    mesh = plsc.VectorSubcoreMesh(
        core_axis_name="c", subcore_axis_name="s", num_cores=NC,
        num_subcores=NS)
    f = pl.kernel(
        _sc_body,
        out_type=[
            jax.ShapeDtypeStruct((2 * NPAD, D), jnp.float32),
            jax.ShapeDtypeStruct((N + 16, D), jnp.float32),
        ],
        mesh=mesh,
        scratch_types=[
            pltpu.VMEM_SHARED((NPAD, D), jnp.float32),
            pltpu.VMEM((2 * GK, D), jnp.int32),
            pltpu.VMEM((2 * K, D), jnp.float32),
            pltpu.VMEM((4, GK), jnp.int32),
            pltpu.VMEM((4, K), jnp.int32),
            pltpu.VMEM((K,), jnp.int32),
            pltpu.VMEM((D // 2,), jnp.int32),
            pltpu.VMEM((16,), jnp.float32),
            pltpu.VMEM((D,), jnp.float32),
            pltpu.SemaphoreType.DMA((4,)),
            pltpu.SemaphoreType.DMA((2,)),
            pltpu.SemaphoreType.DMA((2,)),
        ],
    )
    return f(idxp, dstp, tall, w2b, b2v, entp, ball, ipp)


def _heads_body(hb, sb, pb, wc, ws_, wd, bc2, bs2, bd2, pc, ps, pd):
    hcb = hb[...]
    hsb = sb[...]
    hdb = hcb + pb[...]
    pc[...] = jnp.dot(hcb, wc[...], preferred_element_type=jnp.float32) + bc2[...]
    ps[...] = jnp.dot(hsb, ws_[...], preferred_element_type=jnp.float32) + bs2[...]
    pd[...] = jnp.dot(hdb, wd[...], preferred_element_type=jnp.float32) + bd2[...]


def _heads(hb, sb, pb, wc, ws_, wd, bc2, bs2, bd2):
    BM, BN = 512, 2048
    grid = (pl.cdiv(N, BN), pl.cdiv(N, BM))
    hspec = pl.BlockSpec((BM, D), lambda ni, mj: (mj, 0))
    wspec = pl.BlockSpec((D, BN), lambda ni, mj: (0, ni))
    bspec = pl.BlockSpec((1, BN), lambda ni, mj: (0, ni))
    ospec = pl.BlockSpec((BM, BN), lambda ni, mj: (mj, ni))
    oshape = jax.ShapeDtypeStruct((N, N), jnp.float32)
    return pl.pallas_call(
        _heads_body,
        grid=grid,
        in_specs=[hspec, hspec, hspec, wspec, wspec, wspec, bspec, bspec,
                  bspec],
        out_specs=[ospec, ospec, ospec],
        out_shape=[oshape, oshape, oshape],
    )(hb, sb, pb, wc, ws_, wd, bc2, bs2, bd2)


def kernel(edge_index, edge_type, edge_time, query_rel, entity_emb_c,
           rel_emb_c, time_emb_c, Wc, bc, entity_emb_s, rel_emb_s, time_emb_s,
           Ws, bs, W1, b1, W2, b2, Wpc, bpc, Wps, bps, Wpdo, bpdo):
    f32 = jnp.float32
    i32 = jnp.int32
    src = jnp.asarray(edge_index[0], i32)
    dst = jnp.asarray(edge_index[1], i32)
    typ = jnp.asarray(edge_type, i32)
    tim = jnp.asarray(edge_time, i32)
    pad = EP - E
    srcp = jnp.concatenate([src, jnp.zeros((pad,), i32)]).reshape(NCHT, K)
    dstp = jnp.concatenate([dst, jnp.full((pad,), N, i32)]).reshape(NCHT, K)
    typp = jnp.concatenate([typ, jnp.zeros((pad,), i32)]).reshape(NCHT, K)
    timp = jnp.concatenate([tim, jnp.zeros((pad,), i32)]).reshape(NCHT, K)
    ga_c = jnp.stack(
        [srcp, 2 * N + typp, 2 * N + 2 * R + timp], 1).reshape(NCHT, GK)
    ga_s = jnp.stack(
        [N + srcp, 2 * N + R + typp, 2 * N + 2 * R + TPAD + timp],
        1).reshape(NCHT, GK)
    idxp = jnp.stack([ga_c, ga_s], 0)

    W1a, W1b, W1c, W1d = W1[:D], W1[D:2 * D], W1[2 * D:3 * D], W1[3 * D:]
    rq = lax.dynamic_slice(rel_emb_c, (query_rel, 0), (1, D))
    c08 = _mm(jnp.broadcast_to(rq, (8, D)), W1c)
    c0v = c08[0] + b1

    TEc = _mm(entity_emb_c, jnp.concatenate([W1a, Wc], 1))
    TEs = _mm(jnp.concatenate([entity_emb_c, entity_emb_s], 1),
              _blockdiag(W1a, Ws))
    TRc = _mm(rel_emb_c, jnp.concatenate([W1b, Wc], 1))
    TRs = _mm(jnp.concatenate([rel_emb_c, rel_emb_s], 1), _blockdiag(W1b, Ws))
    tcp = jnp.concatenate([time_emb_c, jnp.zeros((TPAD - 365, D), f32)], 0)
    tsp = jnp.concatenate([time_emb_s, jnp.zeros((TPAD - 365, D), f32)], 0)
    ones = jnp.ones((TPAD, 1), f32)
    z127 = jnp.zeros((TPAD, 127), f32)
    zrow = jnp.zeros((127, 2 * D), f32)
    c0row = jnp.concatenate([c0v, jnp.zeros((D,), f32)]).reshape(1, 2 * D)
    Wtc = jnp.concatenate(
        [jnp.concatenate([W1d, Wc], 1), c0row, zrow], 0)
    TTc = _mm(jnp.concatenate([tcp, ones, z127], 1), Wtc)
    Wts = jnp.concatenate([_blockdiag(W1d, Ws), c0row, zrow], 0)
    TTs = _mm(jnp.concatenate([tcp, tsp, ones, z127], 1), Wts)
    tall = _ileave(jnp.concatenate([TEc, TEs, TRc, TRs, TTc, TTs], 0))

    w2b = _ileave(W2[:, 0].reshape(1, D))[0]
    b2v = jnp.full((16,), b2[0], f32)
    zpad = jnp.zeros((NPAD - N, D), f32)
    entp = lax.bitcast_convert_type(
        jnp.concatenate([entity_emb_c, zpad, entity_emb_s, zpad], 0),
        jnp.int32)
    ball = jnp.stack([bc, bs], 0)

    perm = jax.random.permutation(jax.random.key(42), N)
    inv = jnp.zeros((N,), i32).at[perm].set(jnp.arange(N, dtype=i32))
    ipp = jnp.concatenate([inv, jnp.full((NPAD - N,), N, i32)])

    hall, hsp = _sc_edge(idxp, dstp, tall, w2b, b2v, entp, ball, ipp)
    hc = hall[:N]
    hs = hall[NPAD:NPAD + N]
    hs_perm = hsp[:N]

    bf16 = jnp.bfloat16
    pc, ps, pdo = _heads(
        hc.astype(bf16), hs.astype(bf16), hs_perm.astype(bf16),
        Wpc.astype(bf16), Wps.astype(bf16), Wpdo.astype(bf16),
        bpc.reshape(1, N), bps.reshape(1, N), bpdo.reshape(1, N))
    return (pc, ps, pdo, hc, hs)
